# baseline (speedup 1.0000x reference)
import functools

import jax
import jax.numpy as jnp
from jax import lax
from jax.experimental import pallas as pl
from jax.experimental.pallas import tpu as pltpu
from jax.experimental.pallas import tpu_sc as plsc

_L = 16
_TS = 8
_NOB = 4
_NSC = 16


def _make_lookup(n_trows, n_cols, n_workers):
    cols_per_w = n_cols // n_workers
    mesh = plsc.VectorSubcoreMesh(core_axis_name="c", subcore_axis_name="s")

    @functools.partial(
        pl.kernel,
        mesh=mesh,
        out_type=jax.ShapeDtypeStruct((n_trows, _TS, n_cols), jnp.int32),
        scratch_types=[
            pltpu.VMEM((_L,), jnp.int32),
            pltpu.VMEM((2, _L), jnp.int32),
            pltpu.VMEM((2, _L), jnp.int32),
            pltpu.VMEM_SHARED((_NSC * n_trows, _TS, cols_per_w), jnp.int32),
            [pltpu.VMEM((1, _TS, cols_per_w), jnp.int32) for _ in range(2)],
            [pltpu.VMEM((1, _TS, cols_per_w), jnp.int32)
             for _ in range(_NOB)],
            pltpu.SemaphoreType.DMA,
            [pltpu.SemaphoreType.DMA for _ in range(2)],
            [pltpu.SemaphoreType.DMA for _ in range(_NOB)],
        ],
        compiler_params=pltpu.CompilerParams(needs_layout_passes=False),
    )
    def lookup(idx_hbm, vals_hbm, out_hbm, tab_v, rid_v, sid_v, spm_in,
               in_b, out_b, h2s_sem, s2t_sem, out_sem):
        cid = lax.axis_index("c")
        sid = lax.axis_index("s")
        wid = sid * 2 + cid
        pltpu.sync_copy(vals_hbm, tab_v)
        col0 = wid * cols_per_w

        lane = lax.iota(jnp.int32, _L)
        for j in range(2):
            t = lane + (j * _L + wid)
            t = jnp.where(t >= n_trows, t - n_trows, t)
            t = jnp.where(t >= n_trows, t - n_trows, t)
            rid_v[j, :] = t
            sid_v[j, :] = t + sid * n_trows

        def rid_at(c):
            return rid_v.at[c // _L, pl.ds(c % _L, 1)]

        def sid_at(c):
            return sid_v.at[c // _L, pl.ds(c % _L, 1)]

        def compute(src, dst):
            @plsc.parallel_loop(0, _TS * (cols_per_w // _L), unroll=8)
            def _(v):
                s = v // (cols_per_w // _L)
                k = v % (cols_per_w // _L)
                sl = pl.ds(k * _L, _L)
                dst[0, s, sl] = plsc.load_gather(tab_v, [src[0, s, sl]])

        pltpu.async_copy(
            idx_hbm.at[:, :, pl.ds(col0, cols_per_w)],
            spm_in.at[pl.ds(sid * n_trows, n_trows)], h2s_sem).wait()

        def trow(c):
            t = c + wid
            t = jnp.where(t >= n_trows, t - n_trows, t)
            return jnp.where(t >= n_trows, t - n_trows, t)

        def start_s2t(c, b):
            return pltpu.async_copy(
                spm_in.at[pl.ds(sid * n_trows + trow(c), 1)],
                in_b[b], s2t_sem[b])

        in_cp = [None, None]
        out_cp = [None] * _NOB
        in_cp[0] = start_s2t(0, 0)
        for c in range(n_trows):
            b = c % 2
            ob = c % _NOB
            if c + 1 < n_trows:
                nb = (c + 1) % 2
                in_cp[nb] = start_s2t(c + 1, nb)
            in_cp[b].wait()
            if c >= _NOB:
                out_cp[ob].wait()
            compute(in_b[b], out_b[ob])
            out_cp[ob] = pltpu.async_copy(
                out_b[ob], out_hbm.at[rid_at(c), :, pl.ds(col0, cols_per_w)],
                out_sem[ob])
        for c in range(max(0, n_trows - _NOB), n_trows):
            out_cp[c % _NOB].wait()

    return lookup


def kernel(inputs, vals):
    n_rows, n_cols = inputs.shape
    x = inputs.astype(jnp.int32).T.reshape(n_cols // _TS, _TS, n_rows)
    tab = jnp.pad(vals.astype(jnp.int32), (0, _L - vals.shape[0]))
    out = _make_lookup(n_cols // _TS, n_rows, 32)(x, tab)
    return out.reshape(n_cols, n_rows).T


# speedup vs baseline: 1.2556x; 1.2556x over previous
import functools

import jax
import jax.numpy as jnp
from jax import lax
from jax.experimental import pallas as pl
from jax.experimental.pallas import tpu as pltpu
from jax.experimental.pallas import tpu_sc as plsc

_L = 16
_TS = 8
_NOB = 4


def _make_lookup(n_trows, n_cols, n_workers):
    cols_per_w = n_cols // n_workers
    mesh = plsc.VectorSubcoreMesh(core_axis_name="c", subcore_axis_name="s")

    @functools.partial(
        pl.kernel,
        mesh=mesh,
        out_type=jax.ShapeDtypeStruct((n_trows, _TS, n_cols), jnp.int32),
        scratch_types=[
            pltpu.VMEM((_L,), jnp.int32),
            pltpu.VMEM((2, _L), jnp.int32),
            pltpu.VMEM((n_trows, _TS, cols_per_w), jnp.int32),
            [pltpu.VMEM((1, _TS, cols_per_w), jnp.int32)
             for _ in range(_NOB)],
            pltpu.SemaphoreType.DMA,
            [pltpu.SemaphoreType.DMA for _ in range(_NOB)],
        ],
        compiler_params=pltpu.CompilerParams(needs_layout_passes=False),
    )
    def lookup(idx_hbm, vals_hbm, out_hbm, tab_v, rid_v, in_b, out_b,
               in_sem, out_sem):
        wid = lax.axis_index("s") * 2 + lax.axis_index("c")
        pltpu.sync_copy(vals_hbm, tab_v)
        col0 = wid * cols_per_w

        lane = lax.iota(jnp.int32, _L)
        for j in range(2):
            t = lane + (j * _L + wid)
            t = jnp.where(t >= n_trows, t - n_trows, t)
            t = jnp.where(t >= n_trows, t - n_trows, t)
            rid_v[j, :] = t

        def rid_at(c):
            return rid_v.at[c // _L, pl.ds(c % _L, 1)]

        def compute(c, dst):
            @plsc.parallel_loop(0, _TS * (cols_per_w // _L), unroll=8)
            def _(v):
                s = v // (cols_per_w // _L)
                k = v % (cols_per_w // _L)
                sl = pl.ds(k * _L, _L)
                dst[0, s, sl] = plsc.load_gather(tab_v, [in_b[c, s, sl]])

        in_cp = [
            pltpu.async_copy(
                idx_hbm.at[rid_at(c), :, pl.ds(col0, cols_per_w)],
                in_b.at[pl.ds(c, 1)], in_sem)
            for c in range(n_trows)
        ]
        out_cp = [None] * _NOB
        for c in range(n_trows):
            ob = c % _NOB
            in_cp[c].wait()
            if c >= _NOB:
                out_cp[ob].wait()
            compute(c, out_b[ob])
            out_cp[ob] = pltpu.async_copy(
                out_b[ob], out_hbm.at[rid_at(c), :, pl.ds(col0, cols_per_w)],
                out_sem[ob])
        for c in range(max(0, n_trows - _NOB), n_trows):
            out_cp[c % _NOB].wait()

    return lookup


def kernel(inputs, vals):
    n_rows, n_cols = inputs.shape
    x = inputs.astype(jnp.int32).T.reshape(n_cols // _TS, _TS, n_rows)
    tab = jnp.pad(vals.astype(jnp.int32), (0, _L - vals.shape[0]))
    out = _make_lookup(n_cols // _TS, n_rows, 32)(x, tab)
    return out.reshape(n_cols, n_rows).T
